# baseline (speedup 1.0000x reference)
import functools

import jax
import jax.numpy as jnp
from jax import lax
from jax.experimental import pallas as pl
from jax.experimental.pallas import tpu as pltpu
from jax.experimental.pallas import tpu_sc as plsc

_NUM_CORES = 2
_NUM_SUBCORES = 16
_NUM_WORKERS = _NUM_CORES * _NUM_SUBCORES


def kernel(labels, embedding):
    (B,) = labels.shape
    V, D = embedding.shape
    assert D == _NUM_WORKERS

    mesh = plsc.VectorSubcoreMesh(core_axis_name="c", subcore_axis_name="s")

    @functools.partial(
        pl.kernel,
        mesh=mesh,
        out_type=jax.ShapeDtypeStruct((D, B), jnp.float32),
        scratch_types=[
            pltpu.VMEM((B,), jnp.int32),
            pltpu.VMEM((B,), jnp.float32),
            pltpu.SemaphoreType.DMA,
        ],
        compiler_params=pltpu.CompilerParams(use_tc_tiling_on_sc=False),
    )
    def embed(labels_hbm, table_hbm, out_hbm, idx_v, col_v, sem):
        w = lax.axis_index("s") * _NUM_CORES + lax.axis_index("c")
        pltpu.sync_copy(labels_hbm, idx_v)
        pltpu.async_copy(table_hbm.at[w].at[idx_v], col_v, sem).wait()
        pltpu.sync_copy(col_v, out_hbm.at[w])

    out_t = embed(labels.astype(jnp.int32), embedding.T)
    return out_t.T


# speedup vs baseline: 19.2662x; 19.2662x over previous
import functools

import jax
import jax.numpy as jnp
from jax import lax
from jax.experimental import pallas as pl
from jax.experimental.pallas import tpu as pltpu
from jax.experimental.pallas import tpu_sc as plsc

_NUM_CORES = 2
_NUM_SUBCORES = 16
_NUM_WORKERS = _NUM_CORES * _NUM_SUBCORES
_RING = 8


def kernel(labels, embedding):
    (B,) = labels.shape
    V, D = embedding.shape
    b_per_w = B // _NUM_WORKERS
    n_rounds = b_per_w // _RING
    assert n_rounds % 2 == 0

    mesh = plsc.VectorSubcoreMesh(core_axis_name="c", subcore_axis_name="s")

    @functools.partial(
        pl.kernel,
        mesh=mesh,
        out_type=jax.ShapeDtypeStruct((D, B), jnp.float32),
        scratch_types=[
            pltpu.VMEM((b_per_w,), jnp.int32),
            pltpu.VMEM((2 * _RING, D, 128), jnp.float32),
            pltpu.VMEM((D, b_per_w), jnp.float32),
            pltpu.SemaphoreType.DMA,
            pltpu.SemaphoreType.DMA,
        ],
        compiler_params=pltpu.CompilerParams(
            disable_bounds_checks=True, needs_layout_passes=False
        ),
    )
    def embed(labels_hbm, table_hbm, out_hbm, idx_v, ring_v, blk_v, sem0, sem1):
        w = lax.axis_index("s") * _NUM_CORES + lax.axis_index("c")
        base = w * b_per_w
        pltpu.sync_copy(labels_hbm.at[pl.ds(base, b_per_w)], idx_v)

        lanes_lo = lax.iota(jnp.int32, 16)
        lanes_hi = lanes_lo + 16
        sems = (sem0, sem1)

        def label_scalar(r, k):
            vb = idx_v[pl.ds((r // 2) * 16, 16)]
            lane = (r % 2) * _RING + k
            return jnp.sum(jnp.where(lanes_lo == lane, vb, 0))

        def issue_round(r, half):
            for k in range(_RING):
                c = label_scalar(r, k)
                col = pl.multiple_of((c >> 7) << 7, 128)
                pltpu.async_copy(
                    table_hbm.at[:, pl.ds(col, 128)],
                    ring_v.at[half * _RING + k],
                    sems[half],
                )

        def drain_round(half):
            for k in range(_RING):
                pltpu.make_async_copy(
                    table_hbm.at[:, pl.ds(0, 128)],
                    ring_v.at[half * _RING + k],
                    sems[half],
                ).wait()

        def extract_round(r, half):
            for k in range(_RING):
                j = r * _RING + k
                c = label_scalar(r, k)
                lane = jnp.full((16,), 1, jnp.int32) * (c & 127)
                jcol = jnp.full((16,), 1, jnp.int32) * j
                buf = ring_v.at[half * _RING + k]
                lo = plsc.load_gather(buf, [lanes_lo, lane])
                hi = plsc.load_gather(buf, [lanes_hi, lane])
                plsc.store_scatter(blk_v, [lanes_lo, jcol], lo)
                plsc.store_scatter(blk_v, [lanes_hi, jcol], hi)

        issue_round(0, 0)

        def body(i, carry):
            r0 = 2 * i
            r1 = r0 + 1
            issue_round(r1, 1)
            drain_round(0)
            extract_round(r0, 0)

            @pl.when(r1 + 1 < n_rounds)
            def _():
                issue_round(r1 + 1, 0)

            drain_round(1)
            extract_round(r1, 1)
            return carry

        lax.fori_loop(0, n_rounds // 2, body, 0, unroll=False)
        pltpu.sync_copy(blk_v, out_hbm.at[:, pl.ds(base, b_per_w)])

    out_t = embed(labels.astype(jnp.int32), embedding.T)
    return out_t.T
